# baseline (speedup 1.0000x reference)
import functools

import jax
import jax.numpy as jnp
from jax import lax
from jax.experimental import pallas as pl
from jax.experimental.pallas import tpu as pltpu
from jax.experimental.pallas import tpu_sc as plsc

B, S, H = 1, 2048, 1024
E, TOPK, DFF = 8, 2, 1024
T = B * S
TILE = 128
NT = 40
NROWS = NT * TILE

NC, NS = 2, 16
NW = NC * NS
TPW = T // NW
CHUNK = 32


def _pack_bf16(x):
    n = x.shape[1] // 2
    vb = x.astype(jnp.bfloat16).astype(jnp.float32)
    ui = lax.bitcast_convert_type(vb, jnp.uint32)
    lo = lax.shift_right_logical(ui[:, :n], jnp.uint32(16))
    hi = jnp.bitwise_and(ui[:, n:], jnp.uint32(0xFFFF0000))
    return lax.bitcast_convert_type(jnp.bitwise_or(lo, hi), jnp.int32)


def _unpack_bf16(px):
    pu = lax.bitcast_convert_type(px, jnp.uint32)
    lo = lax.shift_left(pu, jnp.uint32(16))
    hi = jnp.bitwise_and(pu, jnp.uint32(0xFFFF0000))
    return jnp.concatenate(
        [lax.bitcast_convert_type(lo, jnp.float32),
         lax.bitcast_convert_type(hi, jnp.float32)], axis=1)


def _gate_body(x_ref, gk_ref, pos0_ref, pos1_ref, w0_ref, w1_ref, eid_ref,
               xi_ref):
    x = x_ref[...]
    gk = gk_ref[...]
    logits = lax.dot_general(
        x, gk, (((1,), (0,)), ((), ())), preferred_element_type=jnp.float32
    )

    cols = lax.broadcasted_iota(jnp.int32, (T, E), 1)
    m1 = jnp.max(logits, axis=1, keepdims=True)
    idx1 = jnp.min(jnp.where(logits == m1, cols, E), axis=1, keepdims=True)
    masked = jnp.where(cols == idx1, -jnp.inf, logits)
    m2 = jnp.max(masked, axis=1, keepdims=True)
    idx2 = jnp.min(jnp.where(masked == m2, cols, E), axis=1, keepdims=True)
    r = jnp.exp(m2 - m1)
    w0_ref[...] = 1.0 / (1.0 + r)
    w1_ref[...] = r / (1.0 + r)

    oh1 = (cols == idx1).astype(jnp.float32)
    oh2 = (cols == idx2).astype(jnp.float32)
    n_pair = oh1 + oh2
    ri = lax.broadcasted_iota(jnp.int32, (T, T), 0)
    ci = lax.broadcasted_iota(jnp.int32, (T, T), 1)
    tri = (ri >= ci).astype(jnp.bfloat16)
    c_incl = lax.dot_general(
        tri, n_pair.astype(jnp.bfloat16), (((1,), (0,)), ((), ())),
        preferred_element_type=jnp.float32,
    )
    c_excl = c_incl - n_pair
    counts = c_incl[T - 1 : T, :]
    ntiles = jnp.floor((counts + jnp.float32(TILE - 1)) * jnp.float32(1.0 / TILE))
    eri = lax.broadcasted_iota(jnp.int32, (E, E), 0)
    eci = lax.broadcasted_iota(jnp.int32, (E, E), 1)
    stri = (eri < eci).astype(jnp.float32)
    starts = lax.dot_general(
        ntiles, stri, (((1,), (0,)), ((), ())), preferred_element_type=jnp.float32
    )
    row_start = starts * jnp.float32(TILE)

    pos0 = jnp.sum(oh1 * (row_start + c_excl), axis=1, keepdims=True)
    pos1 = jnp.sum(oh2 * (row_start + c_excl), axis=1, keepdims=True)
    pos0_ref[...] = pos0.astype(jnp.int32)
    pos1_ref[...] = pos1.astype(jnp.int32)

    ti = lax.broadcasted_iota(jnp.int32, (1, NT), 1).astype(jnp.float32)
    acc = jnp.zeros((1, NT), jnp.float32)
    for e in range(E):
        acc = acc + (ti >= starts[:, e : e + 1]).astype(jnp.float32)
    eid = jnp.clip(acc - 1.0, 0.0, float(E - 1))
    eid_ref[...] = eid.astype(jnp.int32)

    xi_ref[...] = _pack_bf16(x)


def _scatter_body(flat_hbm, pos0_hbm, pos1_hbm, xpad_hbm,
                  rows_v, idx0_v, idx1_v, sem):
    wid = lax.axis_index("s") * NC + lax.axis_index("c")
    base = wid * TPW
    pltpu.sync_copy(pos0_hbm.at[pl.ds(base, TPW)], idx0_v)
    pltpu.sync_copy(pos1_hbm.at[pl.ds(base, TPW)], idx1_v)
    pltpu.sync_copy(flat_hbm.at[pl.ds(base, TPW)], rows_v)
    c0 = pltpu.async_copy(rows_v, xpad_hbm.at[idx0_v], sem)
    c1 = pltpu.async_copy(rows_v, xpad_hbm.at[idx1_v], sem)
    c0.wait()
    c1.wait()


def _mlp_body(eid_ref, x_ref, w1_ref, b1_ref, w2_ref, b2_ref, out_ref):
    x = _unpack_bf16(x_ref[...])
    h = lax.dot_general(
        x, w1_ref[0], (((1,), (0,)), ((), ())), preferred_element_type=jnp.float32
    )
    h = jax.nn.gelu(h + b1_ref[0])
    o = lax.dot_general(
        h, w2_ref[0], (((1,), (0,)), ((), ())), preferred_element_type=jnp.float32
    )
    out_ref[...] = _pack_bf16(o + b2_ref[0])


def _gather_body(opad_hbm, pos0_hbm, pos1_hbm,
                 g0_hbm, g1_hbm, rows0_v, rows1_v, idx0_v, idx1_v, sem):
    wid = lax.axis_index("s") * NC + lax.axis_index("c")
    for c in range(TPW // CHUNK):
        base = wid * TPW + c * CHUNK
        pltpu.sync_copy(pos0_hbm.at[pl.ds(base, CHUNK)], idx0_v)
        pltpu.sync_copy(pos1_hbm.at[pl.ds(base, CHUNK)], idx1_v)
        c0 = pltpu.async_copy(opad_hbm.at[idx0_v], rows0_v, sem)
        c1 = pltpu.async_copy(opad_hbm.at[idx1_v], rows1_v, sem)
        c0.wait()
        c1.wait()
        pltpu.sync_copy(rows0_v, g0_hbm.at[pl.ds(base, CHUNK)])
        pltpu.sync_copy(rows1_v, g1_hbm.at[pl.ds(base, CHUNK)])


def _shared_body(x_ref, w1_ref, b1_ref, w2_ref, b2_ref, out_ref):
    x = x_ref[...]
    h = lax.dot_general(
        x, w1_ref[...], (((1,), (0,)), ((), ())), preferred_element_type=jnp.float32
    )
    h = jax.nn.gelu(h + b1_ref[...])
    o = lax.dot_general(
        h, w2_ref[...], (((1,), (0,)), ((), ())), preferred_element_type=jnp.float32
    )
    out_ref[...] = o + b2_ref[...]


def _epilogue_body(s_ref, g0_ref, g1_ref, wt0_ref, wt1_ref, out_ref):
    g0 = _unpack_bf16(g0_ref[...])
    g1 = _unpack_bf16(g1_ref[...])
    out_ref[...] = s_ref[...] + wt0_ref[...] * g0 + wt1_ref[...] * g1


def kernel(hidden_states, gate_kernel, W1, b1, W2, b2, Ws1, bs1, Ws2, bs2):
    flat = hidden_states.reshape(T, H)
    gk_t = gate_kernel.T

    pos0, pos1, w0, w1, eid, flat_i = pl.pallas_call(
        _gate_body,
        out_shape=(
            jax.ShapeDtypeStruct((T, 1), jnp.int32),
            jax.ShapeDtypeStruct((T, 1), jnp.int32),
            jax.ShapeDtypeStruct((T, 1), jnp.float32),
            jax.ShapeDtypeStruct((T, 1), jnp.float32),
            jax.ShapeDtypeStruct((1, NT), jnp.int32),
            jax.ShapeDtypeStruct((T, H // 2), jnp.int32),
        ),
    )(flat, gk_t)
    pos0 = pos0.reshape(T)
    pos1 = pos1.reshape(T)
    eid = eid.reshape(NT)

    ST = 256
    shared_out = pl.pallas_call(
        _shared_body,
        grid=(T // ST,),
        in_specs=[
            pl.BlockSpec((ST, H), lambda i: (i, 0)),
            pl.BlockSpec((H, DFF), lambda i: (0, 0)),
            pl.BlockSpec((1, DFF), lambda i: (0, 0)),
            pl.BlockSpec((DFF, H), lambda i: (0, 0)),
            pl.BlockSpec((1, H), lambda i: (0, 0)),
        ],
        out_specs=pl.BlockSpec((ST, H), lambda i: (i, 0)),
        out_shape=jax.ShapeDtypeStruct((T, H), jnp.float32),
    )(flat, Ws1, bs1.reshape(1, DFF), Ws2, bs2.reshape(1, H))

    mesh = plsc.VectorSubcoreMesh(core_axis_name="c", subcore_axis_name="s")
    x_pad = pl.kernel(
        _scatter_body,
        out_type=jax.ShapeDtypeStruct((NROWS, H // 2), jnp.int32),
        mesh=mesh,
        scratch_types=[
            pltpu.VMEM((TPW, H // 2), jnp.int32),
            pltpu.VMEM((TPW,), jnp.int32),
            pltpu.VMEM((TPW,), jnp.int32),
            pltpu.SemaphoreType.DMA,
        ],
    )(flat_i, pos0, pos1)

    b1r = b1.reshape(E, 1, DFF)
    b2r = b2.reshape(E, 1, H)
    o_pad = pl.pallas_call(
        _mlp_body,
        grid_spec=pltpu.PrefetchScalarGridSpec(
            num_scalar_prefetch=1,
            grid=(NT,),
            in_specs=[
                pl.BlockSpec((TILE, H // 2), lambda i, eid_ref: (i, 0)),
                pl.BlockSpec((1, H, DFF), lambda i, eid_ref: (eid_ref[i], 0, 0)),
                pl.BlockSpec((1, 1, DFF), lambda i, eid_ref: (eid_ref[i], 0, 0)),
                pl.BlockSpec((1, DFF, H), lambda i, eid_ref: (eid_ref[i], 0, 0)),
                pl.BlockSpec((1, 1, H), lambda i, eid_ref: (eid_ref[i], 0, 0)),
            ],
            out_specs=pl.BlockSpec((TILE, H // 2), lambda i, eid_ref: (i, 0)),
        ),
        out_shape=jax.ShapeDtypeStruct((NROWS, H // 2), jnp.int32),
    )(eid, x_pad, W1, b1r, W2, b2r)

    g0, g1 = pl.kernel(
        _gather_body,
        out_type=(
            jax.ShapeDtypeStruct((T, H // 2), jnp.int32),
            jax.ShapeDtypeStruct((T, H // 2), jnp.int32),
        ),
        mesh=mesh,
        scratch_types=[
            pltpu.VMEM((CHUNK, H // 2), jnp.int32),
            pltpu.VMEM((CHUNK, H // 2), jnp.int32),
            pltpu.VMEM((CHUNK,), jnp.int32),
            pltpu.VMEM((CHUNK,), jnp.int32),
            pltpu.SemaphoreType.DMA,
        ],
    )(o_pad, pos0, pos1)

    ST = 256
    y = pl.pallas_call(
        _epilogue_body,
        grid=(T // ST,),
        in_specs=[
            pl.BlockSpec((ST, H), lambda i: (i, 0)),
            pl.BlockSpec((ST, H // 2), lambda i: (i, 0)),
            pl.BlockSpec((ST, H // 2), lambda i: (i, 0)),
            pl.BlockSpec((ST, 1), lambda i: (i, 0)),
            pl.BlockSpec((ST, 1), lambda i: (i, 0)),
        ],
        out_specs=pl.BlockSpec((ST, H), lambda i: (i, 0)),
        out_shape=jax.ShapeDtypeStruct((T, H), jnp.float32),
    )(shared_out, g0, g1, w0, w1)

    return y.reshape(B, S, H)


# speedup vs baseline: 1.0105x; 1.0105x over previous
import functools

import jax
import jax.numpy as jnp
from jax import lax
from jax.experimental import pallas as pl
from jax.experimental.pallas import tpu as pltpu
from jax.experimental.pallas import tpu_sc as plsc

B, S, H = 1, 2048, 1024
E, TOPK, DFF = 8, 2, 1024
T = B * S
TILE = 128
NT = 40
NROWS = NT * TILE

NC, NS = 2, 16
NW = NC * NS
TPW = T // NW
CHUNK = 32


def _pack_bf16(x):
    n = x.shape[1] // 2
    vb = x.astype(jnp.bfloat16).astype(jnp.float32)
    ui = lax.bitcast_convert_type(vb, jnp.uint32)
    lo = lax.shift_right_logical(ui[:, :n], jnp.uint32(16))
    hi = jnp.bitwise_and(ui[:, n:], jnp.uint32(0xFFFF0000))
    return lax.bitcast_convert_type(jnp.bitwise_or(lo, hi), jnp.int32)


def _unpack_bf16(px):
    pu = lax.bitcast_convert_type(px, jnp.uint32)
    lo = lax.shift_left(pu, jnp.uint32(16))
    hi = jnp.bitwise_and(pu, jnp.uint32(0xFFFF0000))
    return jnp.concatenate(
        [lax.bitcast_convert_type(lo, jnp.float32),
         lax.bitcast_convert_type(hi, jnp.float32)], axis=1)


def _gate_body(x_ref, gk_ref, pos0_ref, pos1_ref, w0_ref, w1_ref, eid_ref,
               xi_ref):
    x = x_ref[...]
    gk = gk_ref[...]
    logits = lax.dot_general(
        x, gk, (((1,), (0,)), ((), ())), preferred_element_type=jnp.float32
    )

    cols = lax.broadcasted_iota(jnp.int32, (T, E), 1)
    m1 = jnp.max(logits, axis=1, keepdims=True)
    idx1 = jnp.min(jnp.where(logits == m1, cols, E), axis=1, keepdims=True)
    masked = jnp.where(cols == idx1, -jnp.inf, logits)
    m2 = jnp.max(masked, axis=1, keepdims=True)
    idx2 = jnp.min(jnp.where(masked == m2, cols, E), axis=1, keepdims=True)
    r = jnp.exp(m2 - m1)
    w0_ref[...] = 1.0 / (1.0 + r)
    w1_ref[...] = r / (1.0 + r)

    oh1 = (cols == idx1).astype(jnp.float32)
    oh2 = (cols == idx2).astype(jnp.float32)
    n_pair = oh1 + oh2
    ri = lax.broadcasted_iota(jnp.int32, (T, T), 0)
    ci = lax.broadcasted_iota(jnp.int32, (T, T), 1)
    tri = (ri >= ci).astype(jnp.bfloat16)
    c_incl = lax.dot_general(
        tri, n_pair.astype(jnp.bfloat16), (((1,), (0,)), ((), ())),
        preferred_element_type=jnp.float32,
    )
    c_excl = c_incl - n_pair
    counts = c_incl[T - 1 : T, :]
    ntiles = jnp.floor((counts + jnp.float32(TILE - 1)) * jnp.float32(1.0 / TILE))
    eri = lax.broadcasted_iota(jnp.int32, (E, E), 0)
    eci = lax.broadcasted_iota(jnp.int32, (E, E), 1)
    stri = (eri < eci).astype(jnp.float32)
    starts = lax.dot_general(
        ntiles, stri, (((1,), (0,)), ((), ())), preferred_element_type=jnp.float32
    )
    row_start = starts * jnp.float32(TILE)

    pos0 = jnp.sum(oh1 * (row_start + c_excl), axis=1, keepdims=True)
    pos1 = jnp.sum(oh2 * (row_start + c_excl), axis=1, keepdims=True)
    pos0_ref[...] = pos0.astype(jnp.int32)
    pos1_ref[...] = pos1.astype(jnp.int32)

    ti = lax.broadcasted_iota(jnp.int32, (1, NT), 1).astype(jnp.float32)
    acc = jnp.zeros((1, NT), jnp.float32)
    for e in range(E):
        acc = acc + (ti >= starts[:, e : e + 1]).astype(jnp.float32)
    eid = jnp.clip(acc - 1.0, 0.0, float(E - 1))
    eid_ref[...] = eid.astype(jnp.int32)

    xi_ref[...] = _pack_bf16(x)


def _scatter_body(flat_hbm, pos0_hbm, pos1_hbm, xpad_hbm,
                  rows_v, idx0_v, idx1_v, sem):
    wid = lax.axis_index("s") * NC + lax.axis_index("c")
    base = wid * TPW
    pltpu.sync_copy(pos0_hbm.at[pl.ds(base, TPW)], idx0_v)
    pltpu.sync_copy(pos1_hbm.at[pl.ds(base, TPW)], idx1_v)
    pltpu.sync_copy(flat_hbm.at[pl.ds(base, TPW)], rows_v)
    c0 = pltpu.async_copy(rows_v, xpad_hbm.at[idx0_v], sem)
    c1 = pltpu.async_copy(rows_v, xpad_hbm.at[idx1_v], sem)
    c0.wait()
    c1.wait()


def _mlp_body(eid_ref, x_ref, w1_ref, b1_ref, w2_ref, b2_ref, out_ref):
    x = _unpack_bf16(x_ref[...])
    h = lax.dot_general(
        x, w1_ref[0], (((1,), (0,)), ((), ())), preferred_element_type=jnp.float32
    )
    h = jax.nn.gelu(h + b1_ref[0])
    o = lax.dot_general(
        h, w2_ref[0], (((1,), (0,)), ((), ())), preferred_element_type=jnp.float32
    )
    out_ref[...] = _pack_bf16(o + b2_ref[0])


def _gather_body(opad_hbm, pos0_hbm, pos1_hbm,
                 g0_hbm, g1_hbm, rows0_v, rows1_v, idx0_v, idx1_v, sem):
    wid = lax.axis_index("s") * NC + lax.axis_index("c")
    for c in range(TPW // CHUNK):
        base = wid * TPW + c * CHUNK
        pltpu.sync_copy(pos0_hbm.at[pl.ds(base, CHUNK)], idx0_v)
        pltpu.sync_copy(pos1_hbm.at[pl.ds(base, CHUNK)], idx1_v)
        c0 = pltpu.async_copy(opad_hbm.at[idx0_v], rows0_v, sem)
        c1 = pltpu.async_copy(opad_hbm.at[idx1_v], rows1_v, sem)
        c0.wait()
        c1.wait()
        pltpu.sync_copy(rows0_v, g0_hbm.at[pl.ds(base, CHUNK)])
        pltpu.sync_copy(rows1_v, g1_hbm.at[pl.ds(base, CHUNK)])


def _shared_body(x_ref, w1_ref, b1_ref, w2_ref, b2_ref,
                 g0_ref, g1_ref, wt0_ref, wt1_ref, out_ref):
    x = x_ref[...]
    h = lax.dot_general(
        x, w1_ref[...], (((1,), (0,)), ((), ())), preferred_element_type=jnp.float32
    )
    h = jax.nn.gelu(h + b1_ref[...])
    o = lax.dot_general(
        h, w2_ref[...], (((1,), (0,)), ((), ())), preferred_element_type=jnp.float32
    )
    g0 = _unpack_bf16(g0_ref[...])
    g1 = _unpack_bf16(g1_ref[...])
    out_ref[...] = (o + b2_ref[...] + wt0_ref[...] * g0
                    + wt1_ref[...] * g1)


def kernel(hidden_states, gate_kernel, W1, b1, W2, b2, Ws1, bs1, Ws2, bs2):
    flat = hidden_states.reshape(T, H)
    gk_t = gate_kernel.T

    pos0, pos1, w0, w1, eid, flat_i = pl.pallas_call(
        _gate_body,
        out_shape=(
            jax.ShapeDtypeStruct((T, 1), jnp.int32),
            jax.ShapeDtypeStruct((T, 1), jnp.int32),
            jax.ShapeDtypeStruct((T, 1), jnp.float32),
            jax.ShapeDtypeStruct((T, 1), jnp.float32),
            jax.ShapeDtypeStruct((1, NT), jnp.int32),
            jax.ShapeDtypeStruct((T, H // 2), jnp.int32),
        ),
    )(flat, gk_t)
    pos0 = pos0.reshape(T)
    pos1 = pos1.reshape(T)
    eid = eid.reshape(NT)

    mesh = plsc.VectorSubcoreMesh(core_axis_name="c", subcore_axis_name="s")
    x_pad = pl.kernel(
        _scatter_body,
        out_type=jax.ShapeDtypeStruct((NROWS, H // 2), jnp.int32),
        mesh=mesh,
        scratch_types=[
            pltpu.VMEM((TPW, H // 2), jnp.int32),
            pltpu.VMEM((TPW,), jnp.int32),
            pltpu.VMEM((TPW,), jnp.int32),
            pltpu.SemaphoreType.DMA,
        ],
    )(flat_i, pos0, pos1)

    b1r = b1.reshape(E, 1, DFF)
    b2r = b2.reshape(E, 1, H)
    o_pad = pl.pallas_call(
        _mlp_body,
        grid_spec=pltpu.PrefetchScalarGridSpec(
            num_scalar_prefetch=1,
            grid=(NT,),
            in_specs=[
                pl.BlockSpec((TILE, H // 2), lambda i, eid_ref: (i, 0)),
                pl.BlockSpec((1, H, DFF), lambda i, eid_ref: (eid_ref[i], 0, 0)),
                pl.BlockSpec((1, 1, DFF), lambda i, eid_ref: (eid_ref[i], 0, 0)),
                pl.BlockSpec((1, DFF, H), lambda i, eid_ref: (eid_ref[i], 0, 0)),
                pl.BlockSpec((1, 1, H), lambda i, eid_ref: (eid_ref[i], 0, 0)),
            ],
            out_specs=pl.BlockSpec((TILE, H // 2), lambda i, eid_ref: (i, 0)),
        ),
        out_shape=jax.ShapeDtypeStruct((NROWS, H // 2), jnp.int32),
    )(eid, x_pad, W1, b1r, W2, b2r)

    g0, g1 = pl.kernel(
        _gather_body,
        out_type=(
            jax.ShapeDtypeStruct((T, H // 2), jnp.int32),
            jax.ShapeDtypeStruct((T, H // 2), jnp.int32),
        ),
        mesh=mesh,
        scratch_types=[
            pltpu.VMEM((CHUNK, H // 2), jnp.int32),
            pltpu.VMEM((CHUNK, H // 2), jnp.int32),
            pltpu.VMEM((CHUNK,), jnp.int32),
            pltpu.VMEM((CHUNK,), jnp.int32),
            pltpu.SemaphoreType.DMA,
        ],
    )(o_pad, pos0, pos1)

    ST = 256
    y = pl.pallas_call(
        _shared_body,
        grid=(T // ST,),
        in_specs=[
            pl.BlockSpec((ST, H), lambda i: (i, 0)),
            pl.BlockSpec((H, DFF), lambda i: (0, 0)),
            pl.BlockSpec((1, DFF), lambda i: (0, 0)),
            pl.BlockSpec((DFF, H), lambda i: (0, 0)),
            pl.BlockSpec((1, H), lambda i: (0, 0)),
            pl.BlockSpec((ST, H // 2), lambda i: (i, 0)),
            pl.BlockSpec((ST, H // 2), lambda i: (i, 0)),
            pl.BlockSpec((ST, 1), lambda i: (i, 0)),
            pl.BlockSpec((ST, 1), lambda i: (i, 0)),
        ],
        out_specs=pl.BlockSpec((ST, H), lambda i: (i, 0)),
        out_shape=jax.ShapeDtypeStruct((T, H), jnp.float32),
    )(flat, Ws1, bs1.reshape(1, DFF), Ws2, bs2.reshape(1, H), g0, g1, w0, w1)

    return y.reshape(B, S, H)


# speedup vs baseline: 1.0203x; 1.0097x over previous
import jax
import jax.numpy as jnp
from jax import lax
from jax.experimental import pallas as pl
from jax.experimental.pallas import tpu as pltpu
from jax.experimental.pallas import tpu_sc as plsc

B, S, H = 1, 2048, 1024
E, TOPK, DFF = 8, 2, 1024
T = B * S
TILE = 128
NT = 40
NROWS = NT * TILE

NC, NS = 2, 16
NW = NC * NS
TPW = T // NW
CHUNK = 64


def _pack_bf16(x):
    n = x.shape[1] // 2
    vb = x.astype(jnp.bfloat16).astype(jnp.float32)
    ui = lax.bitcast_convert_type(vb, jnp.uint32)
    lo = lax.shift_right_logical(ui[:, :n], jnp.uint32(16))
    hi = jnp.bitwise_and(ui[:, n:], jnp.uint32(0xFFFF0000))
    return lax.bitcast_convert_type(jnp.bitwise_or(lo, hi), jnp.int32)


def _unpack_bf16(px):
    pu = lax.bitcast_convert_type(px, jnp.uint32)
    lo = lax.shift_left(pu, jnp.uint32(16))
    hi = jnp.bitwise_and(pu, jnp.uint32(0xFFFF0000))
    return jnp.concatenate(
        [lax.bitcast_convert_type(lo, jnp.float32),
         lax.bitcast_convert_type(hi, jnp.float32)], axis=1)


def _gate_body(x_ref, gk_ref, pos0_ref, pos1_ref, w0_ref, w1_ref, eid_ref,
               xi_ref):
    x = x_ref[...]
    gk = gk_ref[...]
    logits = lax.dot_general(
        x, gk, (((1,), (0,)), ((), ())), preferred_element_type=jnp.float32
    )

    cols = lax.broadcasted_iota(jnp.int32, (T, E), 1)
    m1 = jnp.max(logits, axis=1, keepdims=True)
    idx1 = jnp.min(jnp.where(logits == m1, cols, E), axis=1, keepdims=True)
    masked = jnp.where(cols == idx1, -jnp.inf, logits)
    m2 = jnp.max(masked, axis=1, keepdims=True)
    idx2 = jnp.min(jnp.where(masked == m2, cols, E), axis=1, keepdims=True)
    r = jnp.exp(m2 - m1)
    w0_ref[...] = 1.0 / (1.0 + r)
    w1_ref[...] = r / (1.0 + r)

    oh1 = (cols == idx1).astype(jnp.float32)
    oh2 = (cols == idx2).astype(jnp.float32)
    n_pair = oh1 + oh2
    ri = lax.broadcasted_iota(jnp.int32, (T, T), 0)
    ci = lax.broadcasted_iota(jnp.int32, (T, T), 1)
    tri = (ri >= ci).astype(jnp.bfloat16)
    c_incl = lax.dot_general(
        tri, n_pair.astype(jnp.bfloat16), (((1,), (0,)), ((), ())),
        preferred_element_type=jnp.float32,
    )
    c_excl = c_incl - n_pair
    counts = c_incl[T - 1 : T, :]
    ntiles = jnp.floor((counts + jnp.float32(TILE - 1)) * jnp.float32(1.0 / TILE))
    eri = lax.broadcasted_iota(jnp.int32, (E, E), 0)
    eci = lax.broadcasted_iota(jnp.int32, (E, E), 1)
    stri = (eri < eci).astype(jnp.float32)
    starts = lax.dot_general(
        ntiles, stri, (((1,), (0,)), ((), ())), preferred_element_type=jnp.float32
    )
    row_start = starts * jnp.float32(TILE)

    pos0 = jnp.sum(oh1 * (row_start + c_excl), axis=1, keepdims=True)
    pos1 = jnp.sum(oh2 * (row_start + c_excl), axis=1, keepdims=True)
    pos0_ref[...] = pos0.astype(jnp.int32)
    pos1_ref[...] = pos1.astype(jnp.int32)

    ti = lax.broadcasted_iota(jnp.int32, (1, NT), 1).astype(jnp.float32)
    acc = jnp.zeros((1, NT), jnp.float32)
    for e in range(E):
        acc = acc + (ti >= starts[:, e : e + 1]).astype(jnp.float32)
    eid = jnp.clip(acc - 1.0, 0.0, float(E - 1))
    eid_ref[...] = eid.astype(jnp.int32)

    xi_ref[...] = _pack_bf16(x)


def _scatter_body(flat_hbm, pos0_hbm, pos1_hbm, xpad_hbm,
                  rows_v, idx0_v, idx1_v, sem):
    wid = lax.axis_index("s") * NC + lax.axis_index("c")
    base = wid * TPW
    pltpu.sync_copy(pos0_hbm.at[pl.ds(base, TPW)], idx0_v)
    pltpu.sync_copy(pos1_hbm.at[pl.ds(base, TPW)], idx1_v)
    pltpu.sync_copy(flat_hbm.at[pl.ds(base, TPW)], rows_v)
    c0 = pltpu.async_copy(rows_v, xpad_hbm.at[idx0_v], sem)
    c1 = pltpu.async_copy(rows_v, xpad_hbm.at[idx1_v], sem)
    c0.wait()
    c1.wait()


def _mlp_body(eid_ref, x_ref, w1_ref, b1_ref, w2_ref, b2_ref, out_ref):
    x = _unpack_bf16(x_ref[...])
    h = lax.dot_general(
        x, w1_ref[0], (((1,), (0,)), ((), ())), preferred_element_type=jnp.float32
    )
    h = jax.nn.gelu(h + b1_ref[0])
    o = lax.dot_general(
        h, w2_ref[0], (((1,), (0,)), ((), ())), preferred_element_type=jnp.float32
    )
    out_ref[...] = _pack_bf16(o + b2_ref[0])


def _gather_body(opad_hbm, pos0_hbm, pos1_hbm,
                 g0_hbm, g1_hbm, rows0_v, rows1_v, idx0_v, idx1_v, sem):
    wid = lax.axis_index("s") * NC + lax.axis_index("c")
    for c in range(TPW // CHUNK):
        base = wid * TPW + c * CHUNK
        pltpu.sync_copy(pos0_hbm.at[pl.ds(base, CHUNK)], idx0_v)
        pltpu.sync_copy(pos1_hbm.at[pl.ds(base, CHUNK)], idx1_v)
        c0 = pltpu.async_copy(opad_hbm.at[idx0_v], rows0_v, sem)
        c1 = pltpu.async_copy(opad_hbm.at[idx1_v], rows1_v, sem)
        c0.wait()
        c1.wait()
        pltpu.sync_copy(rows0_v, g0_hbm.at[pl.ds(base, CHUNK)])
        pltpu.sync_copy(rows1_v, g1_hbm.at[pl.ds(base, CHUNK)])


def _shared_body(x_ref, w1_ref, b1_ref, w2_ref, b2_ref,
                 g0_ref, g1_ref, wt0_ref, wt1_ref, out_ref):
    x = x_ref[...]
    h = lax.dot_general(
        x, w1_ref[...], (((1,), (0,)), ((), ())), preferred_element_type=jnp.float32
    )
    h = jax.nn.gelu(h + b1_ref[...])
    o = lax.dot_general(
        h, w2_ref[...], (((1,), (0,)), ((), ())), preferred_element_type=jnp.float32
    )
    g0 = _unpack_bf16(g0_ref[...])
    g1 = _unpack_bf16(g1_ref[...])
    out_ref[...] = (o + b2_ref[...] + wt0_ref[...] * g0
                    + wt1_ref[...] * g1)


def kernel(hidden_states, gate_kernel, W1, b1, W2, b2, Ws1, bs1, Ws2, bs2):
    flat = hidden_states.reshape(T, H)
    gk_t = gate_kernel.T

    pos0, pos1, w0, w1, eid, flat_i = pl.pallas_call(
        _gate_body,
        out_shape=(
            jax.ShapeDtypeStruct((T, 1), jnp.int32),
            jax.ShapeDtypeStruct((T, 1), jnp.int32),
            jax.ShapeDtypeStruct((T, 1), jnp.float32),
            jax.ShapeDtypeStruct((T, 1), jnp.float32),
            jax.ShapeDtypeStruct((1, NT), jnp.int32),
            jax.ShapeDtypeStruct((T, H // 2), jnp.int32),
        ),
    )(flat, gk_t)
    pos0 = pos0.reshape(T)
    pos1 = pos1.reshape(T)
    eid = eid.reshape(NT)

    mesh = plsc.VectorSubcoreMesh(core_axis_name="c", subcore_axis_name="s")
    x_pad = pl.kernel(
        _scatter_body,
        out_type=jax.ShapeDtypeStruct((NROWS, H // 2), jnp.int32),
        mesh=mesh,
        scratch_types=[
            pltpu.VMEM((TPW, H // 2), jnp.int32),
            pltpu.VMEM((TPW,), jnp.int32),
            pltpu.VMEM((TPW,), jnp.int32),
            pltpu.SemaphoreType.DMA,
        ],
    )(flat_i, pos0, pos1)

    b1r = b1.reshape(E, 1, DFF)
    b2r = b2.reshape(E, 1, H)
    o_pad = pl.pallas_call(
        _mlp_body,
        grid_spec=pltpu.PrefetchScalarGridSpec(
            num_scalar_prefetch=1,
            grid=(NT,),
            in_specs=[
                pl.BlockSpec((TILE, H // 2), lambda i, eid_ref: (i, 0)),
                pl.BlockSpec((1, H, DFF), lambda i, eid_ref: (eid_ref[i], 0, 0)),
                pl.BlockSpec((1, 1, DFF), lambda i, eid_ref: (eid_ref[i], 0, 0)),
                pl.BlockSpec((1, DFF, H), lambda i, eid_ref: (eid_ref[i], 0, 0)),
                pl.BlockSpec((1, 1, H), lambda i, eid_ref: (eid_ref[i], 0, 0)),
            ],
            out_specs=pl.BlockSpec((TILE, H // 2), lambda i, eid_ref: (i, 0)),
        ),
        out_shape=jax.ShapeDtypeStruct((NROWS, H // 2), jnp.int32),
    )(eid, x_pad, W1, b1r, W2, b2r)

    g0, g1 = pl.kernel(
        _gather_body,
        out_type=(
            jax.ShapeDtypeStruct((T, H // 2), jnp.int32),
            jax.ShapeDtypeStruct((T, H // 2), jnp.int32),
        ),
        mesh=mesh,
        scratch_types=[
            pltpu.VMEM((CHUNK, H // 2), jnp.int32),
            pltpu.VMEM((CHUNK, H // 2), jnp.int32),
            pltpu.VMEM((CHUNK,), jnp.int32),
            pltpu.VMEM((CHUNK,), jnp.int32),
            pltpu.SemaphoreType.DMA,
        ],
    )(o_pad, pos0, pos1)

    ST = 256
    y = pl.pallas_call(
        _shared_body,
        grid=(T // ST,),
        in_specs=[
            pl.BlockSpec((ST, H), lambda i: (i, 0)),
            pl.BlockSpec((H, DFF), lambda i: (0, 0)),
            pl.BlockSpec((1, DFF), lambda i: (0, 0)),
            pl.BlockSpec((DFF, H), lambda i: (0, 0)),
            pl.BlockSpec((1, H), lambda i: (0, 0)),
            pl.BlockSpec((ST, H // 2), lambda i: (i, 0)),
            pl.BlockSpec((ST, H // 2), lambda i: (i, 0)),
            pl.BlockSpec((ST, 1), lambda i: (i, 0)),
            pl.BlockSpec((ST, 1), lambda i: (i, 0)),
        ],
        out_specs=pl.BlockSpec((ST, H), lambda i: (i, 0)),
        out_shape=jax.ShapeDtypeStruct((T, H), jnp.float32),
    )(flat, Ws1, bs1.reshape(1, DFF), Ws2, bs2.reshape(1, H), g0, g1, w0, w1)

    return y.reshape(B, S, H)
